# baseline (speedup 1.0000x reference)
import functools

import jax
import jax.numpy as jnp
from jax import lax
from jax.experimental import pallas as pl
from jax.experimental.pallas import tpu as pltpu, tpu_sc as plsc

B_TOTAL = 16384 * 50
D = 64
NW = 32
B_PER_W = B_TOTAL // NW
CH = 512
N_CHUNKS = B_PER_W // CH
IDX_PER_STREAM = 128
STREAMS_PER_CHUNK = CH // IDX_PER_STREAM

_mesh = plsc.VectorSubcoreMesh(core_axis_name="c", subcore_axis_name="s")


@functools.partial(
    pl.kernel,
    out_type=jax.ShapeDtypeStruct((B_TOTAL, D), jnp.float32),
    mesh=_mesh,
    compiler_params=pltpu.CompilerParams(use_tc_tiling_on_sc=False),
    scratch_types=[
        pltpu.VMEM((B_PER_W,), jnp.int32),
        pltpu.VMEM((CH, D), jnp.float32),
        pltpu.VMEM((D,), jnp.float32),
        pltpu.SemaphoreType.DMA,
    ],
)
def _emb_kernel(x_hbm, w_hbm, b_hbm, out_hbm, idx_v, rows_v, bias_v, sem):
    wid = lax.axis_index("s") * 2 + lax.axis_index("c")
    base = wid * B_PER_W

    pltpu.sync_copy(b_hbm, bias_v)
    pltpu.sync_copy(x_hbm.at[pl.ds(base, B_PER_W)], idx_v)

    bias_regs = [bias_v[pl.ds(16 * j, 16)] for j in range(D // 16)]

    def chunk_body(g, carry):
        copies = []
        for j in range(STREAMS_PER_CHUNK):
            copies.append(
                pltpu.async_copy(
                    w_hbm.at[idx_v.at[pl.ds(g * CH + j * IDX_PER_STREAM,
                                            IDX_PER_STREAM)]],
                    rows_v.at[pl.ds(j * IDX_PER_STREAM, IDX_PER_STREAM)],
                    sem,
                )
            )
        for c in copies:
            c.wait()

        def row_body(r, carry2):
            for j in range(D // 16):
                plsc.addupdate(rows_v.at[r, pl.ds(16 * j, 16)], bias_regs[j])
            return carry2

        lax.fori_loop(0, CH, row_body, 0, unroll=4)

        pltpu.sync_copy(rows_v, out_hbm.at[pl.ds(base + g * CH, CH)])
        return carry

    lax.fori_loop(0, N_CHUNKS, chunk_body, 0)


def kernel(x, weight, bias):
    out = _emb_kernel(x.reshape(-1), weight, bias)
    return out.reshape(x.shape[0], x.shape[1], D)
